# baseline (speedup 1.0000x reference)
import functools
import math

import jax
import jax.numpy as jnp
from jax import lax
from jax.experimental import pallas as pl
from jax.experimental.pallas import tpu as pltpu
from jax.experimental.pallas import tpu_sc as plsc

D_MODEL = 1024
MAX_SEQ_LEN = 2048
_SCALE = math.sqrt(D_MODEL)

_NC, _NS, _L = 2, 16, 16
_NW = _NC * _NS
_C = 32


def _sinusoidal_pe(max_seq_len: int, d_model: int) -> jnp.ndarray:
    position = jnp.arange(0, max_seq_len, dtype=jnp.float32)[:, None]
    div_term = jnp.exp(
        jnp.arange(0, d_model, 2, dtype=jnp.float32) * (-math.log(10000.0) / d_model)
    )
    pe = jnp.zeros((max_seq_len, d_model), dtype=jnp.float32)
    pe = pe.at[:, 0::2].set(jnp.sin(position * div_term))
    pe = pe.at[:, 1::2].set(jnp.cos(position * div_term))
    return pe


def _embed(xf, pe, table, *, tok, tpw, nchunk):
    mesh = plsc.VectorSubcoreMesh(core_axis_name="c", subcore_axis_name="s")

    @functools.partial(
        pl.kernel,
        out_type=jax.ShapeDtypeStruct((tok, D_MODEL), jnp.float32),
        mesh=mesh,
        scratch_types=[
            pltpu.VMEM((tpw,), jnp.int32),
            pltpu.VMEM((_C, D_MODEL), jnp.float32),
            pltpu.VMEM((_C, D_MODEL), jnp.float32),
            pltpu.SemaphoreType.DMA,
        ],
    )
    def k(xf_hbm, pe_hbm, table_hbm, out_hbm, idx_v, rows_v, pe_v, sem):
        wid = lax.axis_index("s") * _NC + lax.axis_index("c")
        base = wid * tpw
        pbase = lax.rem(base, MAX_SEQ_LEN)
        pltpu.sync_copy(xf_hbm.at[pl.ds(base, tpw)], idx_v)
        for c in range(nchunk):
            pltpu.sync_copy(pe_hbm.at[pl.ds(pbase + c * _C, _C), :], pe_v)
            pltpu.async_copy(
                table_hbm.at[idx_v.at[pl.ds(c * _C, _C)]], rows_v, sem
            ).wait()

            def row_body(i, _):
                def grp(j, _):
                    g = rows_v[i, pl.ds(j * _L, _L)]
                    p = pe_v[i, pl.ds(j * _L, _L)]
                    rows_v[i, pl.ds(j * _L, _L)] = g * _SCALE + p
                    return 0

                return lax.fori_loop(0, D_MODEL // _L, grp, 0)

            lax.fori_loop(0, _C, row_body, 0)
            pltpu.sync_copy(rows_v, out_hbm.at[pl.ds(base + c * _C, _C), :])

    return k(xf, pe, table)


def kernel(x, table):
    b, s = x.shape
    tok = b * s
    tpw = tok // _NW
    nchunk = tpw // _C
    pe = _sinusoidal_pe(MAX_SEQ_LEN, D_MODEL)[:s]
    xf = x.reshape(tok).astype(jnp.int32)
    out = _embed(xf, pe, table, tok=tok, tpw=tpw, nchunk=nchunk)
    return out.reshape(b, s, D_MODEL)


# speedup vs baseline: 1.9838x; 1.9838x over previous
import functools
import math

import jax
import jax.numpy as jnp
from jax import lax
from jax.experimental import pallas as pl
from jax.experimental.pallas import tpu as pltpu
from jax.experimental.pallas import tpu_sc as plsc

D_MODEL = 1024
MAX_SEQ_LEN = 2048
_SCALE = math.sqrt(D_MODEL)

_NC, _NS, _L = 2, 16, 16
_NW = _NC * _NS
_CP = 16
_NBUF = 3


def _sinusoidal_pe(max_seq_len: int, d_model: int) -> jnp.ndarray:
    position = jnp.arange(0, max_seq_len, dtype=jnp.float32)[:, None]
    div_term = jnp.exp(
        jnp.arange(0, d_model, 2, dtype=jnp.float32) * (-math.log(10000.0) / d_model)
    )
    pe = jnp.zeros((max_seq_len, d_model), dtype=jnp.float32)
    pe = pe.at[:, 0::2].set(jnp.sin(position * div_term))
    pe = pe.at[:, 1::2].set(jnp.cos(position * div_term))
    return pe


def _embed(xf, pe, table, *, b_dim, s):
    ppw = s // _NW
    pcb = ppw // _CP
    nchunk = b_dim * pcb
    mesh = plsc.VectorSubcoreMesh(core_axis_name="c", subcore_axis_name="s")

    @functools.partial(
        pl.kernel,
        out_type=jax.ShapeDtypeStruct((b_dim * s, D_MODEL), jnp.float32),
        mesh=mesh,
        scratch_types=[
            pltpu.VMEM((b_dim * ppw,), jnp.int32),
            pltpu.VMEM((ppw, D_MODEL), jnp.float32),
        ]
        + [pltpu.VMEM((_CP, D_MODEL), jnp.float32) for _ in range(_NBUF)]
        + [pltpu.SemaphoreType.DMA for _ in range(1 + 2 * _NBUF)],
    )
    def k(xf_hbm, pe_hbm, table_hbm, out_hbm, idx_v, pe_v, *rest):
        bufs = rest[:_NBUF]
        pe_sem = rest[_NBUF]
        g_sems = rest[_NBUF + 1 : 2 * _NBUF + 1]
        o_sems = rest[2 * _NBUF + 1 :]
        wid = lax.axis_index("s") * _NC + lax.axis_index("c")
        pbase = wid * ppw

        for b in range(b_dim):
            pltpu.sync_copy(
                xf_hbm.at[pl.ds(b * s + pbase, ppw)], idx_v.at[pl.ds(b * ppw, ppw)]
            )
        pe_dma = pltpu.async_copy(pe_hbm.at[pl.ds(pbase, ppw), :], pe_v, pe_sem)

        def issue_gather(c):
            b, o = divmod(c, pcb)
            return pltpu.async_copy(
                table_hbm.at[idx_v.at[pl.ds(b * ppw + o * _CP, _CP)]],
                bufs[c % _NBUF],
                g_sems[c % _NBUF],
            )

        g_dma = [None] * _NBUF
        out_dma = [None] * _NBUF
        for c in range(min(2, nchunk)):
            g_dma[c % _NBUF] = issue_gather(c)
        pe_dma.wait()

        for c in range(nchunk):
            nb = c % _NBUF
            buf = bufs[nb]
            g_dma[nb].wait()
            if c + 2 < nchunk:
                nb2 = (c + 2) % _NBUF
                if out_dma[nb2] is not None:
                    out_dma[nb2].wait()
                g_dma[nb2] = issue_gather(c + 2)

            b, o = divmod(c, pcb)
            off = o * _CP

            @plsc.parallel_loop(0, _CP * (D_MODEL // _L), 1, unroll=8)
            def _fma(kk):
                i = lax.shift_right_logical(kk, 6)
                j = pl.multiple_of(
                    lax.shift_left(lax.bitwise_and(kk, D_MODEL // _L - 1), 4), _L
                )
                buf[i, pl.ds(j, _L)] = (
                    buf[i, pl.ds(j, _L)] * _SCALE + pe_v[off + i, pl.ds(j, _L)]
                )

            out_dma[nb] = pltpu.async_copy(
                buf, out_hbm.at[pl.ds(b * s + pbase + off, _CP), :], o_sems[nb]
            )
        for nb in range(_NBUF):
            if out_dma[nb] is not None:
                out_dma[nb].wait()

    return k(xf, pe, table)


def kernel(x, table):
    b_dim, s = x.shape
    pe = _sinusoidal_pe(MAX_SEQ_LEN, D_MODEL)[:s]
    xf = x.reshape(b_dim * s).astype(jnp.int32)
    out = _embed(xf, pe, table, b_dim=b_dim, s=s)
    return out.reshape(b_dim, s, D_MODEL)


# speedup vs baseline: 3.3879x; 1.7078x over previous
import functools
import math

import jax
import jax.numpy as jnp
import numpy as np
from jax import lax
from jax.experimental import pallas as pl
from jax.experimental.pallas import tpu as pltpu
from jax.experimental.pallas import tpu_sc as plsc

D_MODEL = 1024
MAX_SEQ_LEN = 2048
_SCALE = math.sqrt(D_MODEL)

_NC, _NS, _L = 2, 16, 16
_NW = _NC * _NS
_CP = 16
_NBUF = 3


def _sinusoidal_pe(max_seq_len: int, d_model: int) -> np.ndarray:
    position = np.arange(0, max_seq_len, dtype=np.float32)[:, None]
    div_term = np.exp(
        np.arange(0, d_model, 2, dtype=np.float32)
        * np.float32(-math.log(10000.0) / d_model)
    ).astype(np.float32)
    pe = np.zeros((max_seq_len, d_model), dtype=np.float32)
    pe[:, 0::2] = np.sin(position * div_term, dtype=np.float32)
    pe[:, 1::2] = np.cos(position * div_term, dtype=np.float32)
    return pe


def _embed(xf, pe, table, *, b_dim, s):
    ppw = s // _NW
    pcb = ppw // _CP
    nchunk = b_dim * pcb
    mesh = plsc.VectorSubcoreMesh(core_axis_name="c", subcore_axis_name="s")

    @functools.partial(
        pl.kernel,
        out_type=jax.ShapeDtypeStruct((b_dim * s, D_MODEL), jnp.float32),
        mesh=mesh,
        scratch_types=[
            pltpu.VMEM((b_dim * ppw,), jnp.int32),
            pltpu.VMEM((ppw, D_MODEL), jnp.float32),
        ]
        + [pltpu.VMEM((_CP, D_MODEL), jnp.float32) for _ in range(_NBUF)]
        + [pltpu.SemaphoreType.DMA for _ in range(1 + 2 * _NBUF)],
    )
    def k(xf_hbm, pe_hbm, table_hbm, out_hbm, idx_v, pe_v, *rest):
        bufs = rest[:_NBUF]
        pe_sem = rest[_NBUF]
        g_sems = rest[_NBUF + 1 : 2 * _NBUF + 1]
        o_sems = rest[2 * _NBUF + 1 :]
        wid = lax.axis_index("s") * _NC + lax.axis_index("c")
        pbase = wid * ppw

        for b in range(b_dim):
            pltpu.sync_copy(
                xf_hbm.at[pl.ds(b * s + pbase, ppw)], idx_v.at[pl.ds(b * ppw, ppw)]
            )
        pe_dma = pltpu.async_copy(pe_hbm.at[pl.ds(pbase, ppw), :], pe_v, pe_sem)

        def issue_gather(c):
            b, o = divmod(c, pcb)
            return pltpu.async_copy(
                table_hbm.at[idx_v.at[pl.ds(b * ppw + o * _CP, _CP)]],
                bufs[c % _NBUF],
                g_sems[c % _NBUF],
            )

        g_dma = [None] * _NBUF
        out_dma = [None] * _NBUF
        for c in range(min(2, nchunk)):
            g_dma[c % _NBUF] = issue_gather(c)
        pe_dma.wait()

        for c in range(nchunk):
            nb = c % _NBUF
            buf = bufs[nb]
            g_dma[nb].wait()
            if c + 2 < nchunk:
                nb2 = (c + 2) % _NBUF
                if out_dma[nb2] is not None:
                    out_dma[nb2].wait()
                g_dma[nb2] = issue_gather(c + 2)

            b, o = divmod(c, pcb)
            off = o * _CP

            @plsc.parallel_loop(0, _CP * (D_MODEL // _L), 1, unroll=8)
            def _fma(kk):
                i = lax.shift_right_logical(kk, 6)
                j = pl.multiple_of(
                    lax.shift_left(lax.bitwise_and(kk, D_MODEL // _L - 1), 4), _L
                )
                buf[i, pl.ds(j, _L)] = (
                    buf[i, pl.ds(j, _L)] * _SCALE + pe_v[off + i, pl.ds(j, _L)]
                )

            out_dma[nb] = pltpu.async_copy(
                buf, out_hbm.at[pl.ds(b * s + pbase + off, _CP), :], o_sems[nb]
            )
        for nb in range(_NBUF):
            if out_dma[nb] is not None:
                out_dma[nb].wait()

    return k(xf, pe, table)


def kernel(x, table):
    b_dim, s = x.shape
    pe = _sinusoidal_pe(MAX_SEQ_LEN, D_MODEL)[:s]
    xf = x.reshape(b_dim * s).astype(jnp.int32)
    out = _embed(xf, pe, table, b_dim=b_dim, s=s)
    return out.reshape(b_dim, s, D_MODEL)


# speedup vs baseline: 3.6835x; 1.0872x over previous
import functools
import math

import jax
import jax.numpy as jnp
import numpy as np
from jax import lax
from jax.experimental import pallas as pl
from jax.experimental.pallas import tpu as pltpu
from jax.experimental.pallas import tpu_sc as plsc

D_MODEL = 1024
MAX_SEQ_LEN = 2048
_SCALE = math.sqrt(D_MODEL)

_NC, _NS, _L = 2, 16, 16
_NW = _NC * _NS
_CP = 16
_NBUF = 5
_AHEAD = 3


def _sinusoidal_pe(max_seq_len: int, d_model: int) -> np.ndarray:
    position = np.arange(0, max_seq_len, dtype=np.float32)[:, None]
    div_term = np.exp(
        np.arange(0, d_model, 2, dtype=np.float32)
        * np.float32(-math.log(10000.0) / d_model)
    ).astype(np.float32)
    pe = np.zeros((max_seq_len, d_model), dtype=np.float32)
    pe[:, 0::2] = np.sin(position * div_term, dtype=np.float32)
    pe[:, 1::2] = np.cos(position * div_term, dtype=np.float32)
    return pe


def _embed(xf, pe, table, *, b_dim, s):
    ppw = s // _NW
    pcb = ppw // _CP
    nchunk = b_dim * pcb
    mesh = plsc.VectorSubcoreMesh(core_axis_name="c", subcore_axis_name="s")

    @functools.partial(
        pl.kernel,
        out_type=jax.ShapeDtypeStruct((b_dim * s, D_MODEL), jnp.float32),
        mesh=mesh,
        scratch_types=[
            pltpu.VMEM((b_dim, ppw), jnp.int32),
            pltpu.VMEM((_CP, D_MODEL), jnp.float32),
            pltpu.VMEM((_CP, D_MODEL), jnp.float32),
        ]
        + [pltpu.VMEM((_CP, D_MODEL), jnp.float32) for _ in range(_NBUF)]
        + [pltpu.SemaphoreType.DMA for _ in range(2 + 2 * _NBUF)],
    )
    def k(xf_hbm, pe_hbm, table_hbm, out_hbm, idx_v, pe_v0, pe_v1, *rest):
        bufs = rest[:_NBUF]
        pe_sems = rest[_NBUF : _NBUF + 2]
        g_sems = rest[_NBUF + 2 : 2 * _NBUF + 2]
        o_sems = rest[2 * _NBUF + 2 :]
        pe_bufs = (pe_v0, pe_v1)
        wid = lax.axis_index("s") * _NC + lax.axis_index("c")
        pbase = wid * ppw

        for b in range(b_dim):
            pltpu.sync_copy(xf_hbm.at[pl.ds(b * s + pbase, ppw)], idx_v.at[b])

        def issue_pe(o):
            return pltpu.async_copy(
                pe_hbm.at[pl.ds(pbase + o * _CP, _CP), :],
                pe_bufs[o % 2],
                pe_sems[o % 2],
            )

        def issue_gather(c):
            o, b = divmod(c, b_dim)
            return pltpu.async_copy(
                table_hbm.at[idx_v.at[b, pl.ds(o * _CP, _CP)]],
                bufs[c % _NBUF],
                g_sems[c % _NBUF],
            )

        pe_dma = [None, None]
        pe_dma[0] = issue_pe(0)
        if pcb > 1:
            pe_dma[1] = issue_pe(1)

        g_dma = [None] * _NBUF
        out_dma = [None] * _NBUF
        for c in range(min(_AHEAD, nchunk)):
            g_dma[c % _NBUF] = issue_gather(c)

        for c in range(nchunk):
            o, b = divmod(c, b_dim)
            nb = c % _NBUF
            buf = bufs[nb]
            if b == 0:
                pe_dma[o % 2].wait()
            pe_v = pe_bufs[o % 2]
            g_dma[nb].wait()
            if c + _AHEAD < nchunk:
                nb2 = (c + _AHEAD) % _NBUF
                if out_dma[nb2] is not None:
                    out_dma[nb2].wait()
                g_dma[nb2] = issue_gather(c + _AHEAD)

            @plsc.parallel_loop(0, _CP * (D_MODEL // _L), 1, unroll=8)
            def _fma(kk):
                i = lax.shift_right_logical(kk, 6)
                j = pl.multiple_of(
                    lax.shift_left(lax.bitwise_and(kk, D_MODEL // _L - 1), 4), _L
                )
                buf[i, pl.ds(j, _L)] = (
                    buf[i, pl.ds(j, _L)] * _SCALE + pe_v[i, pl.ds(j, _L)]
                )

            out_dma[nb] = pltpu.async_copy(
                buf, out_hbm.at[pl.ds(b * s + pbase + o * _CP, _CP), :], o_sems[nb]
            )
            if b == b_dim - 1 and o + 2 < pcb:
                pe_dma[o % 2] = issue_pe(o + 2)
        for nb in range(_NBUF):
            if out_dma[nb] is not None:
                out_dma[nb].wait()

    return k(xf, pe, table)


def kernel(x, table):
    b_dim, s = x.shape
    pe = _sinusoidal_pe(MAX_SEQ_LEN, D_MODEL)[:s]
    xf = x.reshape(b_dim * s).astype(jnp.int32)
    out = _embed(xf, pe, table, b_dim=b_dim, s=s)
    return out.reshape(b_dim, s, D_MODEL)
